# baseline (speedup 1.0000x reference)
import functools

import numpy as np
import jax
import jax.numpy as jnp
from jax import lax
from jax.experimental import pallas as pl
from jax.experimental.pallas import tpu as pltpu
from jax.experimental.pallas import tpu_sc as plsc

_H_SPLIT = np.array([12, 15, 19, 22, 25, 29, 33, 37, 41, 46, 51, 56, 63, 69,
                     76, 85, 97, 116, 148, 100000], dtype=np.int64)
_W_SPLIT = np.array([14, 18, 21, 25, 28, 31, 35, 40, 45, 50, 56, 62, 70, 78,
                     86, 96, 110, 125, 157, 100000], dtype=np.int64)
_MAX_LEVEL = 20
_HALF = 64

_vals = np.arange(256, dtype=np.int64)
_LUT_H = (np.minimum(np.searchsorted(_H_SPLIT, _vals, side="right"),
                     _MAX_LEVEL - 1) * _HALF).astype(np.int32)
_LUT_W = ((np.minimum(np.searchsorted(_W_SPLIT, _vals, side="right"),
                      _MAX_LEVEL - 1) + _MAX_LEVEL) * _HALF).astype(np.int32)
_LUT = np.concatenate([_LUT_H, _LUT_W])

_NW = 32
_CH = 256
_NBUF = 2
_L = 16
_GP = _CH // _L


def _body(emb_hbm, hs_hbm, ws_hbm, lut_hbm, table_hbm, out_hbm,
          table_v, lut_v, hs_all, ws_all,
          *bufs_and_sems, rows_per_worker):
    embs = bufs_and_sems[:_NBUF]
    sin = bufs_and_sems[_NBUF:2 * _NBUF]
    sout = bufs_and_sems[2 * _NBUF:3 * _NBUF]
    wid = lax.axis_index("s") * 2 + lax.axis_index("c")
    row0 = wid * rows_per_worker
    n_chunks = rows_per_worker // _CH
    riota = jnp.arange(_L, dtype=jnp.int32) * 128

    def prologue_copies():
        for src, dst in ((lut_hbm, lut_v), (table_hbm, table_v),
                         (hs_hbm.at[pl.ds(row0, rows_per_worker)], hs_all),
                         (ws_hbm.at[pl.ds(row0, rows_per_worker)], ws_all)):
            pltpu.make_async_copy(src, dst, sin[1]).start()

    def prologue_wait():
        for src, dst in ((lut_hbm, lut_v), (table_hbm, table_v),
                         (hs_hbm.at[pl.ds(row0, rows_per_worker)], hs_all),
                         (ws_hbm.at[pl.ds(row0, rows_per_worker)], ws_all)):
            pltpu.make_async_copy(src, dst, sin[1]).wait()

    def in_copy(ci, b):
        base = (row0 + ci * _CH) * 128
        pltpu.make_async_copy(
            emb_hbm.at[pl.ds(base, _CH * 128)], embs[b], sin[b]).start()

    def in_wait(b):
        pltpu.make_async_copy(
            emb_hbm.at[pl.ds(0, _CH * 128)], embs[b], sin[b]).wait()

    def out_copy(ci, b):
        base = (row0 + ci * _CH) * 128
        pltpu.make_async_copy(
            embs[b], out_hbm.at[pl.ds(base, _CH * 128)], sout[b]).start()

    def out_wait(b):
        pltpu.make_async_copy(
            embs[b], out_hbm.at[pl.ds(0, _CH * 128)], sout[b]).wait()

    def compute(ci, b):
        ev = embs[b]
        diag = jnp.arange(_L, dtype=jnp.int32)

        def group(g, _):
            gg = ci * _GP + g
            sv_h = hs_all[pl.ds(gg * _L, _L)]
            sv_w = ws_all[pl.ds(gg * _L, _L)]
            sv_h = jnp.minimum(jnp.maximum(sv_h, 0), 255)
            sv_w = jnp.minimum(jnp.maximum(sv_w, 0), 255) + 256
            idx_h = plsc.load_gather(lut_v, [sv_h])
            idx_w = plsc.load_gather(lut_v, [sv_w])
            rb = riota + g * (_L * 128)

            @plsc.parallel_loop(0, _HALF, unroll=8)
            def _cols(u):
                d = (diag + u) & (_HALF - 1)
                th = plsc.load_gather(table_v, [idx_h + d])
                plsc.addupdate_scatter(ev, [rb + d], th)
                tw = plsc.load_gather(table_v, [idx_w + d])
                plsc.addupdate_scatter(ev, [rb + (_HALF + d)], tw)

            return 0

        lax.fori_loop(0, _GP, group, 0)

    in_copy(0, 0)
    prologue_copies()
    prologue_wait()

    def outer(o, _):
        for b in range(_NBUF):
            ci = o * _NBUF + b
            bn = (b + 1) % _NBUF

            @pl.when(ci >= _NBUF - 1)
            def _():
                out_wait(bn)

            @pl.when(ci + 1 < n_chunks)
            def _():
                in_copy(ci + 1, bn)

            in_wait(b)
            compute(ci, b)
            out_copy(ci, b)
        return 0

    lax.fori_loop(0, n_chunks // _NBUF, outer, 0)
    for k in range(1, _NBUF):
        out_wait(k)


def kernel(emb, h_scales, w_scales, pe_h, pe_w):
    n, ex, f = emb.shape
    nrows = n * ex
    rows_per_worker = nrows // _NW
    emb_flat = emb.reshape(nrows * f)
    hs = h_scales.reshape(nrows)
    ws = w_scales.reshape(nrows)
    table = jnp.concatenate([pe_h, pe_w], axis=0).reshape(2 * _MAX_LEVEL * _HALF)
    lut = jnp.asarray(_LUT)

    mesh = plsc.VectorSubcoreMesh(core_axis_name="c", subcore_axis_name="s")
    k = pl.kernel(
        functools.partial(_body, rows_per_worker=rows_per_worker),
        mesh=mesh,
        compiler_params=pltpu.CompilerParams(needs_layout_passes=False),
        out_type=jax.ShapeDtypeStruct((nrows * f,), jnp.float32),
        scratch_types=[
            pltpu.VMEM((2 * _MAX_LEVEL * _HALF,), jnp.float32),
            pltpu.VMEM((512,), jnp.int32),
            pltpu.VMEM((nrows // _NW,), jnp.int32),
            pltpu.VMEM((nrows // _NW,), jnp.int32),
        ] + [pltpu.VMEM((_CH * 128,), jnp.float32)] * _NBUF
          + [pltpu.SemaphoreType.DMA] * (2 * _NBUF),
    )
    out = k(emb_flat, hs, ws, lut, table)
    return out.reshape(n, ex, f)


# speedup vs baseline: 1.0054x; 1.0054x over previous
import functools

import numpy as np
import jax
import jax.numpy as jnp
from jax import lax
from jax.experimental import pallas as pl
from jax.experimental.pallas import tpu as pltpu
from jax.experimental.pallas import tpu_sc as plsc

_H_SPLIT = np.array([12, 15, 19, 22, 25, 29, 33, 37, 41, 46, 51, 56, 63, 69,
                     76, 85, 97, 116, 148, 100000], dtype=np.int64)
_W_SPLIT = np.array([14, 18, 21, 25, 28, 31, 35, 40, 45, 50, 56, 62, 70, 78,
                     86, 96, 110, 125, 157, 100000], dtype=np.int64)
_MAX_LEVEL = 20
_HALF = 64

_vals = np.arange(256, dtype=np.int64)
_LUT_H = (np.minimum(np.searchsorted(_H_SPLIT, _vals, side="right"),
                     _MAX_LEVEL - 1) * _HALF).astype(np.int32)
_LUT_W = ((np.minimum(np.searchsorted(_W_SPLIT, _vals, side="right"),
                      _MAX_LEVEL - 1) + _MAX_LEVEL) * _HALF).astype(np.int32)
_LUT = np.concatenate([_LUT_H, _LUT_W])

_NW = 32
_CH = 64
_NBUF = 8
_L = 16
_GP = _CH // _L


def _body(emb_hbm, hs_hbm, ws_hbm, lut_hbm, table_hbm, out_hbm,
          table_v, lut_v, hs_all, ws_all,
          *bufs_and_sems, rows_per_worker):
    embs = bufs_and_sems[:_NBUF]
    sin = bufs_and_sems[_NBUF:2 * _NBUF]
    sout = bufs_and_sems[2 * _NBUF:3 * _NBUF]
    wid = lax.axis_index("s") * 2 + lax.axis_index("c")
    row0 = wid * rows_per_worker
    n_chunks = rows_per_worker // _CH
    riota = jnp.arange(_L, dtype=jnp.int32) * 128

    def prologue_copies():
        for src, dst in ((lut_hbm, lut_v), (table_hbm, table_v),
                         (hs_hbm.at[pl.ds(row0, rows_per_worker)], hs_all),
                         (ws_hbm.at[pl.ds(row0, rows_per_worker)], ws_all)):
            pltpu.make_async_copy(src, dst, sin[1]).start()

    def prologue_wait():
        for src, dst in ((lut_hbm, lut_v), (table_hbm, table_v),
                         (hs_hbm.at[pl.ds(row0, rows_per_worker)], hs_all),
                         (ws_hbm.at[pl.ds(row0, rows_per_worker)], ws_all)):
            pltpu.make_async_copy(src, dst, sin[1]).wait()

    def in_copy(ci, b):
        base = (row0 + ci * _CH) * 128
        pltpu.make_async_copy(
            emb_hbm.at[pl.ds(base, _CH * 128)], embs[b], sin[b]).start()

    def in_wait(b):
        pltpu.make_async_copy(
            emb_hbm.at[pl.ds(0, _CH * 128)], embs[b], sin[b]).wait()

    def out_copy(ci, b):
        base = (row0 + ci * _CH) * 128
        pltpu.make_async_copy(
            embs[b], out_hbm.at[pl.ds(base, _CH * 128)], sout[b]).start()

    def out_wait(b):
        pltpu.make_async_copy(
            embs[b], out_hbm.at[pl.ds(0, _CH * 128)], sout[b]).wait()

    def compute(ci, b):
        ev = embs[b]
        diag = jnp.arange(_L, dtype=jnp.int32)

        def group(g, _):
            gg = ci * _GP + g
            sv_h = hs_all[pl.ds(gg * _L, _L)]
            sv_w = ws_all[pl.ds(gg * _L, _L)]
            sv_h = jnp.minimum(jnp.maximum(sv_h, 0), 255)
            sv_w = jnp.minimum(jnp.maximum(sv_w, 0), 255) + 256
            idx_h = plsc.load_gather(lut_v, [sv_h])
            idx_w = plsc.load_gather(lut_v, [sv_w])
            rb = riota + g * (_L * 128)

            @plsc.parallel_loop(0, _HALF, unroll=8)
            def _cols(u):
                d = (diag + u) & (_HALF - 1)
                th = plsc.load_gather(table_v, [idx_h + d])
                plsc.addupdate_scatter(ev, [rb + d], th)
                tw = plsc.load_gather(table_v, [idx_w + d])
                plsc.addupdate_scatter(ev, [rb + (_HALF + d)], tw)

            return 0

        lax.fori_loop(0, _GP, group, 0)

    in_copy(0, 0)
    prologue_copies()
    prologue_wait()

    def outer(o, _):
        for b in range(_NBUF):
            ci = o * _NBUF + b
            bn = (b + 1) % _NBUF

            @pl.when(ci >= _NBUF - 1)
            def _():
                out_wait(bn)

            @pl.when(ci + 1 < n_chunks)
            def _():
                in_copy(ci + 1, bn)

            in_wait(b)
            compute(ci, b)
            out_copy(ci, b)
        return 0

    lax.fori_loop(0, n_chunks // _NBUF, outer, 0)
    for k in range(1, _NBUF):
        out_wait(k)


def kernel(emb, h_scales, w_scales, pe_h, pe_w):
    n, ex, f = emb.shape
    nrows = n * ex
    rows_per_worker = nrows // _NW
    emb_flat = emb.reshape(nrows * f)
    hs = h_scales.reshape(nrows)
    ws = w_scales.reshape(nrows)
    table = jnp.concatenate([pe_h, pe_w], axis=0).reshape(2 * _MAX_LEVEL * _HALF)
    lut = jnp.asarray(_LUT)

    mesh = plsc.VectorSubcoreMesh(core_axis_name="c", subcore_axis_name="s")
    k = pl.kernel(
        functools.partial(_body, rows_per_worker=rows_per_worker),
        mesh=mesh,
        compiler_params=pltpu.CompilerParams(needs_layout_passes=False),
        out_type=jax.ShapeDtypeStruct((nrows * f,), jnp.float32),
        scratch_types=[
            pltpu.VMEM((2 * _MAX_LEVEL * _HALF,), jnp.float32),
            pltpu.VMEM((512,), jnp.int32),
            pltpu.VMEM((nrows // _NW,), jnp.int32),
            pltpu.VMEM((nrows // _NW,), jnp.int32),
        ] + [pltpu.VMEM((_CH * 128,), jnp.float32)] * _NBUF
          + [pltpu.SemaphoreType.DMA] * (2 * _NBUF),
    )
    out = k(emb_flat, hs, ws, lut, table)
    return out.reshape(n, ex, f)


# speedup vs baseline: 1.1523x; 1.1461x over previous
import functools

import numpy as np
import jax
import jax.numpy as jnp
from jax import lax
from jax.experimental import pallas as pl
from jax.experimental.pallas import tpu as pltpu
from jax.experimental.pallas import tpu_sc as plsc

_H_SPLIT = np.array([12, 15, 19, 22, 25, 29, 33, 37, 41, 46, 51, 56, 63, 69,
                     76, 85, 97, 116, 148, 100000], dtype=np.int64)
_W_SPLIT = np.array([14, 18, 21, 25, 28, 31, 35, 40, 45, 50, 56, 62, 70, 78,
                     86, 96, 110, 125, 157, 100000], dtype=np.int64)
_MAX_LEVEL = 20
_HALF = 64

_vals = np.arange(256, dtype=np.int64)
_LUT_H = (np.minimum(np.searchsorted(_H_SPLIT, _vals, side="right"),
                     _MAX_LEVEL - 1) * _HALF).astype(np.int32)
_LUT_W = ((np.minimum(np.searchsorted(_W_SPLIT, _vals, side="right"),
                      _MAX_LEVEL - 1) + _MAX_LEVEL) * _HALF).astype(np.int32)
_LUT = np.concatenate([_LUT_H, _LUT_W])

_NW = 32
_CH = 128
_NBUF = 4
_L = 16
_GP = _CH // _L


def _body(emb_hbm, hs_hbm, ws_hbm, lut_hbm, table_hbm, out_hbm,
          table_v, lut_v, hs_all, ws_all,
          *bufs_and_sems, rows_per_worker):
    embs = bufs_and_sems[:_NBUF]
    sin = bufs_and_sems[_NBUF:2 * _NBUF]
    sout = bufs_and_sems[2 * _NBUF:3 * _NBUF]
    wid = lax.axis_index("s") * 2 + lax.axis_index("c")
    row0 = wid * rows_per_worker
    n_chunks = rows_per_worker // _CH
    riota = jnp.arange(_L, dtype=jnp.int32) * 128

    def prologue_copies():
        for src, dst in ((lut_hbm, lut_v), (table_hbm, table_v),
                         (hs_hbm.at[pl.ds(row0, rows_per_worker)], hs_all),
                         (ws_hbm.at[pl.ds(row0, rows_per_worker)], ws_all)):
            pltpu.make_async_copy(src, dst, sin[1]).start()

    def prologue_wait():
        for src, dst in ((lut_hbm, lut_v), (table_hbm, table_v),
                         (hs_hbm.at[pl.ds(row0, rows_per_worker)], hs_all),
                         (ws_hbm.at[pl.ds(row0, rows_per_worker)], ws_all)):
            pltpu.make_async_copy(src, dst, sin[1]).wait()

    def in_copy(ci, b):
        base = (row0 + ci * _CH) * 128
        pltpu.make_async_copy(
            emb_hbm.at[pl.ds(base, _CH * 128)], embs[b], sin[b]).start()

    def in_wait(b):
        pltpu.make_async_copy(
            emb_hbm.at[pl.ds(0, _CH * 128)], embs[b], sin[b]).wait()

    def out_copy(ci, b):
        base = (row0 + ci * _CH) * 128
        pltpu.make_async_copy(
            embs[b], out_hbm.at[pl.ds(base, _CH * 128)], sout[b]).start()

    def out_wait(b):
        pltpu.make_async_copy(
            embs[b], out_hbm.at[pl.ds(0, _CH * 128)], sout[b]).wait()

    def compute(ci, b):
        ev = embs[b]
        diag = jnp.arange(_L, dtype=jnp.int32)

        def group(g, _):
            gg = ci * _GP + g
            sv_h = hs_all[pl.ds(gg * _L, _L)]
            sv_w = ws_all[pl.ds(gg * _L, _L)]
            sv_h = jnp.minimum(jnp.maximum(sv_h, 0), 255)
            sv_w = jnp.minimum(jnp.maximum(sv_w, 0), 255) + 256
            idx_h = plsc.load_gather(lut_v, [sv_h])
            idx_w = plsc.load_gather(lut_v, [sv_w])
            rb = riota + g * (_L * 128)

            @plsc.parallel_loop(0, _HALF, unroll=16)
            def _cols(u):
                d = (diag + u) & (_HALF - 1)
                th = plsc.load_gather(table_v, [idx_h + d])
                plsc.addupdate_scatter(ev, [rb + d], th)
                tw = plsc.load_gather(table_v, [idx_w + d])
                plsc.addupdate_scatter(ev, [rb + (_HALF + d)], tw)

            return 0

        lax.fori_loop(0, _GP, group, 0)

    in_copy(0, 0)
    prologue_copies()
    prologue_wait()

    def outer(o, _):
        for b in range(_NBUF):
            ci = o * _NBUF + b
            bn = (b + 1) % _NBUF

            @pl.when(ci >= _NBUF - 1)
            def _():
                out_wait(bn)

            @pl.when(ci + 1 < n_chunks)
            def _():
                in_copy(ci + 1, bn)

            in_wait(b)
            compute(ci, b)
            out_copy(ci, b)
        return 0

    lax.fori_loop(0, n_chunks // _NBUF, outer, 0)
    for k in range(1, _NBUF):
        out_wait(k)


def kernel(emb, h_scales, w_scales, pe_h, pe_w):
    n, ex, f = emb.shape
    nrows = n * ex
    rows_per_worker = nrows // _NW
    emb_flat = emb.reshape(nrows * f)
    hs = h_scales.reshape(nrows)
    ws = w_scales.reshape(nrows)
    table = jnp.concatenate([pe_h, pe_w], axis=0).reshape(2 * _MAX_LEVEL * _HALF)
    lut = jnp.asarray(_LUT)

    mesh = plsc.VectorSubcoreMesh(core_axis_name="c", subcore_axis_name="s")
    k = pl.kernel(
        functools.partial(_body, rows_per_worker=rows_per_worker),
        mesh=mesh,
        compiler_params=pltpu.CompilerParams(needs_layout_passes=False),
        out_type=jax.ShapeDtypeStruct((nrows * f,), jnp.float32),
        scratch_types=[
            pltpu.VMEM((2 * _MAX_LEVEL * _HALF,), jnp.float32),
            pltpu.VMEM((512,), jnp.int32),
            pltpu.VMEM((nrows // _NW,), jnp.int32),
            pltpu.VMEM((nrows // _NW,), jnp.int32),
        ] + [pltpu.VMEM((_CH * 128,), jnp.float32)] * _NBUF
          + [pltpu.SemaphoreType.DMA] * (2 * _NBUF),
    )
    out = k(emb_flat, hs, ws, lut, table)
    return out.reshape(n, ex, f)


# speedup vs baseline: 1.1722x; 1.0173x over previous
import functools

import numpy as np
import jax
import jax.numpy as jnp
from jax import lax
from jax.experimental import pallas as pl
from jax.experimental.pallas import tpu as pltpu
from jax.experimental.pallas import tpu_sc as plsc

_H_SPLIT = np.array([12, 15, 19, 22, 25, 29, 33, 37, 41, 46, 51, 56, 63, 69,
                     76, 85, 97, 116, 148, 100000], dtype=np.int64)
_W_SPLIT = np.array([14, 18, 21, 25, 28, 31, 35, 40, 45, 50, 56, 62, 70, 78,
                     86, 96, 110, 125, 157, 100000], dtype=np.int64)
_MAX_LEVEL = 20
_HALF = 64

_vals = np.arange(256, dtype=np.int64)
_LUT_H = (np.minimum(np.searchsorted(_H_SPLIT, _vals, side="right"),
                     _MAX_LEVEL - 1) * _HALF).astype(np.int32)
_LUT_W = ((np.minimum(np.searchsorted(_W_SPLIT, _vals, side="right"),
                      _MAX_LEVEL - 1) + _MAX_LEVEL) * _HALF).astype(np.int32)
_LUT = np.concatenate([_LUT_H, _LUT_W])

_NW = 32
_CH = 128
_NBUF = 4
_L = 16
_GP = _CH // _L


def _body(emb_hbm, hs_hbm, ws_hbm, lut_hbm, table_hbm, out_hbm,
          table_v, lut_v, hs_all, ws_all,
          *bufs_and_sems, rows_per_worker):
    embs = bufs_and_sems[:_NBUF]
    sin = bufs_and_sems[_NBUF:2 * _NBUF]
    sout = bufs_and_sems[2 * _NBUF:3 * _NBUF]
    wid = lax.axis_index("s") * 2 + lax.axis_index("c")
    row0 = wid * rows_per_worker
    n_chunks = rows_per_worker // _CH
    riota = jnp.arange(_L, dtype=jnp.int32) * 128

    def prologue_copies():
        for src, dst in ((lut_hbm, lut_v), (table_hbm, table_v),
                         (hs_hbm.at[pl.ds(row0, rows_per_worker)], hs_all),
                         (ws_hbm.at[pl.ds(row0, rows_per_worker)], ws_all)):
            pltpu.make_async_copy(src, dst, sin[1]).start()

    def prologue_wait():
        for src, dst in ((lut_hbm, lut_v), (table_hbm, table_v),
                         (hs_hbm.at[pl.ds(row0, rows_per_worker)], hs_all),
                         (ws_hbm.at[pl.ds(row0, rows_per_worker)], ws_all)):
            pltpu.make_async_copy(src, dst, sin[1]).wait()

    def in_copy(ci, b):
        base = (row0 + ci * _CH) * 128
        pltpu.make_async_copy(
            emb_hbm.at[pl.ds(base, _CH * 128)], embs[b], sin[b]).start()

    def in_wait(b):
        pltpu.make_async_copy(
            emb_hbm.at[pl.ds(0, _CH * 128)], embs[b], sin[b]).wait()

    def out_copy(ci, b):
        base = (row0 + ci * _CH) * 128
        pltpu.make_async_copy(
            embs[b], out_hbm.at[pl.ds(base, _CH * 128)], sout[b]).start()

    def out_wait(b):
        pltpu.make_async_copy(
            embs[b], out_hbm.at[pl.ds(0, _CH * 128)], sout[b]).wait()

    def compute(ci, b):
        ev = embs[b]
        diag = jnp.arange(_L, dtype=jnp.int32)

        def group(g, _):
            gg = ci * _GP + g
            sv_h = hs_all[pl.ds(gg * _L, _L)]
            sv_w = ws_all[pl.ds(gg * _L, _L)]
            sv_h = jnp.minimum(jnp.maximum(sv_h, 0), 255)
            sv_w = jnp.minimum(jnp.maximum(sv_w, 0), 255) + 256
            idx_h = plsc.load_gather(lut_v, [sv_h])
            idx_w = plsc.load_gather(lut_v, [sv_w])
            rb = riota + g * (_L * 128)

            @plsc.parallel_loop(0, _HALF, unroll=8)
            def _cols(u):
                d = (diag + u) & (_HALF - 1)
                th = plsc.load_gather(table_v, [idx_h + d])
                plsc.addupdate_scatter(ev, [rb + d], th)
                tw = plsc.load_gather(table_v, [idx_w + d])
                plsc.addupdate_scatter(ev, [rb + (_HALF + d)], tw)

            return 0

        lax.fori_loop(0, _GP, group, 0)

    in_copy(0, 0)
    prologue_copies()
    prologue_wait()

    def outer(o, _):
        for b in range(_NBUF):
            ci = o * _NBUF + b
            bn = (b + 1) % _NBUF

            @pl.when(ci >= _NBUF - 1)
            def _():
                out_wait(bn)

            @pl.when(ci + 1 < n_chunks)
            def _():
                in_copy(ci + 1, bn)

            in_wait(b)
            compute(ci, b)
            out_copy(ci, b)
        return 0

    lax.fori_loop(0, n_chunks // _NBUF, outer, 0)
    for k in range(1, _NBUF):
        out_wait(k)


def kernel(emb, h_scales, w_scales, pe_h, pe_w):
    n, ex, f = emb.shape
    nrows = n * ex
    rows_per_worker = nrows // _NW
    emb_flat = emb.reshape(nrows * f)
    hs = h_scales.reshape(nrows)
    ws = w_scales.reshape(nrows)
    table = jnp.concatenate([pe_h, pe_w], axis=0).reshape(2 * _MAX_LEVEL * _HALF)
    lut = jnp.asarray(_LUT)

    mesh = plsc.VectorSubcoreMesh(core_axis_name="c", subcore_axis_name="s")
    k = pl.kernel(
        functools.partial(_body, rows_per_worker=rows_per_worker),
        mesh=mesh,
        compiler_params=pltpu.CompilerParams(needs_layout_passes=False),
        out_type=jax.ShapeDtypeStruct((nrows * f,), jnp.float32),
        scratch_types=[
            pltpu.VMEM((2 * _MAX_LEVEL * _HALF,), jnp.float32),
            pltpu.VMEM((512,), jnp.int32),
            pltpu.VMEM((nrows // _NW,), jnp.int32),
            pltpu.VMEM((nrows // _NW,), jnp.int32),
        ] + [pltpu.VMEM((_CH * 128,), jnp.float32)] * _NBUF
          + [pltpu.SemaphoreType.DMA] * (2 * _NBUF),
    )
    out = k(emb_flat, hs, ws, lut, table)
    return out.reshape(n, ex, f)
